# baseline (speedup 1.0000x reference)
import jax
import jax.numpy as jnp
from jax import lax
from jax.experimental import pallas as pl
from jax.experimental.pallas import tpu as pltpu

S, B, OBS, D, H, DH = 2048, 1, 256, 768, 12, 64
DFF_BASE, DFF_AD, E, OUT = 2048, 1024, 8, 256
NL = E + 1
G = 16
NC = NL * G

_HI = lax.Precision.HIGHEST
_LO = lax.Precision.DEFAULT

_N_SMALL = 10
_N_BIG = 6


def _dot(a, b, prec=_HI):
    return jnp.dot(a, b, precision=prec, preferred_element_type=jnp.float32)


def _ln_row(u, g, e):
    m = jnp.mean(u, axis=-1, keepdims=True)
    v = jnp.mean((u - m) ** 2, axis=-1, keepdims=True)
    return (u - m) * lax.rsqrt(v + 1e-5) * g + e


def _mega_body(*refs):
    (obs_ref, win_ref, bin_ref, wg1_ref, bg1_ref, wg2_ref, bg2_ref,
     wout_ref, bout_ref, bz_ref) = refs[:10]
    small = [refs[10 + _N_SMALL * l: 10 + _N_SMALL * (l + 1)] for l in range(NL)]
    big0 = 10 + _N_SMALL * NL
    bigw = [refs[big0 + _N_BIG * l: big0 + _N_BIG * (l + 1)] for l in range(NL)]
    wz_ref = refs[big0 + _N_BIG * NL]
    out_ref = refs[big0 + _N_BIG * NL + 1]
    abuf, bbuf, f1buf, f2buf, sem_a, sem_b, sem_f1, sem_f2 = refs[-8:]

    inflight = {}

    def _start_rows(pool_ref, sem_ref, slot, src, rows, nch, scol=None, roff=0):
        cps = inflight.setdefault((id(pool_ref), slot), [])
        r = rows // nch
        for i in range(nch):
            dst = pool_ref.at[slot, i * r:(i + 1) * r, :]
            lo = roff + i * r
            s = (src.at[lo:lo + r, :] if scol is None
                 else src.at[lo:lo + r, scol:scol + DFF_AD])
            cp = pltpu.make_async_copy(s, dst, sem_ref.at[slot])
            cp.start()
            cps.append(cp)

    def wait(pool_ref, slot):
        for cp in inflight.pop((id(pool_ref), slot)):
            cp.wait()

    ab_seq = ([(bigw[l][0], bigw[l][1]) for l in range(NL)]
              + [(bigw[l][2], bigw[l][3]) for l in range(NL)]
              + [(wz_ref.at[e], None) for e in range(E)])

    def issue_ab(k):
        if k >= len(ab_seq):
            return
        slot = k % 4
        src_a, src_b = ab_seq[k]
        _start_rows(abuf, sem_a, slot, src_a, D, 2)
        if src_b is not None:
            _start_rows(bbuf, sem_b, slot, src_b, D, 2)

    f_seq = [(0, 0), (0, 1)] + [(l, 0) for l in range(1, NL)]

    def issue_f(k):
        if k >= len(f_seq):
            return
        l, j = f_seq[k]
        slot = k % 3
        _start_rows(f1buf, sem_f1, slot, bigw[l][4], D, 2, scol=j * DFF_AD)
        _start_rows(f2buf, sem_f2, slot, bigw[l][5], DFF_AD, 2,
                    roff=j * DFF_AD)

    issue_ab(0)
    issue_ab(1)
    issue_ab(2)

    x = _dot(obs_ref[...], win_ref[...]) + bin_ref[...]
    x0 = x[0:1, :]
    h1 = _dot(x0, wg1_ref[0:D, :]) + _dot(x[1:2, :], wg1_ref[D:2 * D, :])
    h1 = jnp.maximum(h1 + bg1_ref[...], 0.0)
    logits = _dot(h1, wg2_ref[...]) + bg2_ref[...]
    lmax = jnp.max(logits, axis=-1, keepdims=True)
    pg = jnp.exp(logits - lmax)
    pg = pg / jnp.sum(pg, axis=-1, keepdims=True)
    idx = lax.broadcasted_iota(jnp.int32, (1, E + 1), 1)
    ks = jnp.min(jnp.where(logits >= lmax, idx, E + 1))
    t_i = lax.broadcasted_iota(jnp.int32, (E + 1, E), 0)
    j_i = lax.broadcasted_iota(jnp.int32, (E + 1, E), 1)
    w = _dot(pg, (t_i >= j_i + 1).astype(jnp.float32))
    i_idx = lax.broadcasted_iota(jnp.int32, (1, E), 1) + 1
    coef = w * (i_idx <= ks).astype(jnp.float32)

    r_i = lax.broadcasted_iota(jnp.int32, (D, NC), 0)
    c_i = lax.broadcasted_iota(jnp.int32, (D, NC), 1)
    m_acc = jnp.zeros((D, NC), jnp.float32)
    bt_acc = jnp.zeros((1, NC), jnp.float32)
    for l in range(NL):
        slot = l % 4
        wait(abuf, slot)
        wait(bbuf, slot)
        issue_ab(l + 3)
        bq, bk = small[l][0], small[l][1]
        q0 = _dot(x0, abuf[slot]) + bq[...]
        seg = (c_i == l * G + r_i // DH).astype(jnp.float32)
        m_acc = m_acc + _dot(bbuf[slot] * q0, seg, _LO)
        bt_acc = bt_acc + _dot(bk[...] * q0, seg, _LO)

    s = (_dot(x, m_acc, _LO) + bt_acc) * (1.0 / 8.0)
    smax = jnp.max(s, axis=0, keepdims=True)
    pexp = jnp.exp(s - smax)
    patt = pexp * (1.0 / jnp.sum(pexp, axis=0, keepdims=True))
    a_all = lax.dot_general(patt, x, (((0,), (0,)), ((), ())),
                            precision=_LO,
                            preferred_element_type=jnp.float32)

    issue_f(0)
    issue_f(1)

    dr = lax.broadcasted_iota(jnp.int32, (H, D), 0)
    dc = lax.broadcasted_iota(jnp.int32, (H, D), 1)
    diag = (dc // DH == dr).astype(jnp.float32)
    x1s = []
    for l in range(NL):
        k = NL + l
        slot = k % 4
        wait(abuf, slot)
        wait(bbuf, slot)
        issue_ab(k + 3)
        bv, bo, g1, e1 = small[l][2], small[l][3], small[l][4], small[l][5]
        a_l = a_all[l * G: l * G + H, :]
        t_full = _dot(a_l, abuf[slot], _LO)
        o0 = jnp.sum(t_full * diag, axis=0, keepdims=True) + bv[...]
        u = x0 + _dot(o0, bbuf[slot], _LO) + bo[...]
        x1s.append(_ln_row(u, g1[...], e1[...]))

    hs = []
    kf = 0
    for l in range(NL):
        nj = 2 if l == 0 else 1
        c1, c2, g2, e2 = small[l][6], small[l][7], small[l][8], small[l][9]
        x1 = x1s[l]
        f_sum = None
        for j in range(nj):
            slot = kf % 3
            wait(f1buf, slot)
            wait(f2buf, slot)
            issue_f(kf + 2)
            c1c = c1[...][j * DFF_AD:(j + 1) * DFF_AD] if nj == 2 else c1[...]
            fmid = jnp.maximum(_dot(x1, f1buf[slot], _LO) + c1c, 0.0)
            part = _dot(fmid, f2buf[slot], _LO)
            f_sum = part if f_sum is None else f_sum + part
            kf += 1
        hs.append(_ln_row(x1 + f_sum + c2[...], g2[...], e2[...]))

    res = jnp.zeros((1, D), jnp.float32)
    for e in range(E):
        k = 2 * NL + e
        slot = k % 4
        wait(abuf, slot)
        issue_ab(k + 3)
        r_e = _dot(hs[e + 1], abuf[slot], _LO) + bz_ref[e: e + 1, :]
        res = res + coef[:, e: e + 1] * r_e
    out_ref[...] = _dot(hs[0] + res, wout_ref[...]) + bout_ref[...]


def _f32(shape):
    return jax.ShapeDtypeStruct(shape, jnp.float32)


def kernel(raw_obs, params):
    p = params
    obs = raw_obs.reshape(S, OBS)
    layers = [p['base']] + list(p['adapters'])

    args = [obs, p['W_in'], p['b_in'], p['Wg1'], p['bg1'],
            p['Wg2'], p['bg2'], p['W_out'], p['b_out'], p['bz']]
    n_vmem_in = len(args) + _N_SMALL * NL
    for lp in layers:
        args += [lp['bq'], lp['bk'], lp['bv'], lp['bo'],
                 lp['g1'], lp['e1'], lp['c1'], lp['c2'],
                 lp['g2'], lp['e2']]
    for lp in layers:
        args += [lp['Wq'], lp['Wk'], lp['Wv'], lp['Wo'], lp['W1'], lp['W2']]
    args.append(p['Wz'])

    in_specs = ([pl.BlockSpec(memory_space=pltpu.MemorySpace.VMEM)] * n_vmem_in
                + [pl.BlockSpec(memory_space=pltpu.MemorySpace.HBM)]
                * (_N_BIG * NL + 1))

    out = pl.pallas_call(
        _mega_body,
        in_specs=in_specs,
        out_shape=_f32((1, OUT)),
        scratch_shapes=[
            pltpu.VMEM((4, D, D), jnp.float32),
            pltpu.VMEM((4, D, D), jnp.float32),
            pltpu.VMEM((3, D, DFF_AD), jnp.float32),
            pltpu.VMEM((3, DFF_AD, D), jnp.float32),
            pltpu.SemaphoreType.DMA((4,)),
            pltpu.SemaphoreType.DMA((4,)),
            pltpu.SemaphoreType.DMA((3,)),
            pltpu.SemaphoreType.DMA((3,)),
        ],
    )(*args)

    return (out, jnp.array(0.0, jnp.float32))
